# baseline (speedup 1.0000x reference)
import jax
import jax.numpy as jnp
from jax.experimental import pallas as pl
from jax.experimental.pallas import tpu as pltpu


def _gcn_block(adj_ref, inp_ref, bw_ref, bc_ref, out_ref, x_ref):
    r_dim, _, n = adj_ref.shape
    n_basis = bw_ref.shape[0]

    @pl.when(pl.program_id(0) == 0)
    def _compute_x():
        for r in range(r_dim):
            w_r = None
            for b in range(n_basis):
                term = bc_ref[r, b] * bw_ref[b]
                w_r = term if w_r is None else w_r + term
            x_ref[r] = jnp.dot(
                inp_ref[...], w_r, preferred_element_type=jnp.float32
            ).astype(jnp.bfloat16)

    acc = None
    for r in range(r_dim):
        part = jnp.dot(
            adj_ref[r].astype(jnp.bfloat16),
            x_ref[r],
            preferred_element_type=jnp.float32,
        )
        acc = part if acc is None else acc + part
    out_ref[...] = acc


def kernel(inp, adj_mat_list, basis_weights, basis_coeff):
    n, in_size = inp.shape
    r_dim = adj_mat_list.shape[0]
    n_basis, _, out_size = basis_weights.shape
    blk = 128

    return pl.pallas_call(
        _gcn_block,
        grid=(n // blk,),
        in_specs=[
            pl.BlockSpec((r_dim, blk, n), lambda i: (0, i, 0)),
            pl.BlockSpec((n, in_size), lambda i: (0, 0)),
            pl.BlockSpec((n_basis, in_size, out_size), lambda i: (0, 0, 0)),
            pl.BlockSpec(memory_space=pltpu.SMEM),
        ],
        out_specs=pl.BlockSpec((blk, out_size), lambda i: (i, 0)),
        out_shape=jax.ShapeDtypeStruct((n, out_size), jnp.float32),
        scratch_shapes=[pltpu.VMEM((r_dim, n, out_size), jnp.bfloat16)],
        compiler_params=pltpu.CompilerParams(
            dimension_semantics=("arbitrary",),
        ),
    )(adj_mat_list, inp, basis_weights, basis_coeff)


# speedup vs baseline: 1.1629x; 1.1629x over previous
import jax
import jax.numpy as jnp
from jax.experimental import pallas as pl
from jax.experimental.pallas import tpu as pltpu


def _gcn_block(adj0_ref, adj1_ref, inp_ref, bw_ref, bc_ref, out_ref, x_ref):
    r_dim = 2
    n_basis = bw_ref.shape[0]

    @pl.when(pl.program_id(0) == 0)
    def _compute_x():
        for r in range(r_dim):
            w_r = None
            for b in range(n_basis):
                term = bc_ref[r, b] * bw_ref[b]
                w_r = term if w_r is None else w_r + term
            x_ref[r] = jnp.dot(
                inp_ref[...], w_r, preferred_element_type=jnp.float32
            ).astype(jnp.bfloat16)

    acc = None
    for r, adj_ref in enumerate((adj0_ref, adj1_ref)):
        part = jnp.dot(
            adj_ref[0].astype(jnp.bfloat16),
            x_ref[r],
            preferred_element_type=jnp.float32,
        )
        acc = part if acc is None else acc + part
    out_ref[...] = acc


def kernel(inp, adj_mat_list, basis_weights, basis_coeff):
    n, in_size = inp.shape
    r_dim = adj_mat_list.shape[0]
    n_basis, _, out_size = basis_weights.shape
    blk = 256

    return pl.pallas_call(
        _gcn_block,
        grid=(n // blk,),
        in_specs=[
            pl.BlockSpec((1, blk, n), lambda i: (0, i, 0)),
            pl.BlockSpec((1, blk, n), lambda i: (1, i, 0)),
            pl.BlockSpec((n, in_size), lambda i: (0, 0)),
            pl.BlockSpec((n_basis, in_size, out_size), lambda i: (0, 0, 0)),
            pl.BlockSpec(memory_space=pltpu.SMEM),
        ],
        out_specs=pl.BlockSpec((blk, out_size), lambda i: (i, 0)),
        out_shape=jax.ShapeDtypeStruct((n, out_size), jnp.float32),
        scratch_shapes=[pltpu.VMEM((r_dim, n, out_size), jnp.bfloat16)],
        compiler_params=pltpu.CompilerParams(
            dimension_semantics=("arbitrary",),
        ),
    )(adj_mat_list, adj_mat_list, inp, basis_weights, basis_coeff)
